# Initial kernel scaffold; baseline (speedup 1.0000x reference)
import jax
import jax.numpy as jnp
from jax.experimental import pallas as pl


def kernel(embeddings_table, length_q, length_k, relative_v):
    raise NotImplementedError("write your pallas kernel here")


# speedup vs baseline: 5.7991x; 5.7991x over previous
import functools

import jax
import jax.numpy as jnp
from jax import lax
from jax.experimental import pallas as pl
from jax.experimental.pallas import tpu as pltpu
from jax.experimental.pallas import tpu_sc as plsc

_NUM_UNITS = 64
_MAX_REL = 128
_L = 2048


def _rel_pos_sc(table, rv):
    rows = table.shape[0]
    rv_arr = jnp.full((16,), rv, dtype=jnp.int32)
    e_rows = 2 * _L
    nc, ns = 2, 16
    nw = nc * ns
    rows_per_tile = e_rows // ns
    out_rows_per_worker = _L // nw

    mesh = plsc.VectorSubcoreMesh(
        core_axis_name="c", subcore_axis_name="s", num_cores=nc, num_subcores=ns
    )

    @functools.partial(
        pl.kernel,
        out_type=jax.ShapeDtypeStruct((_L, _L, _NUM_UNITS), jnp.float32),
        mesh=mesh,
        scratch_types=[
            pltpu.VMEM((16,), jnp.int32),
            pltpu.VMEM((rows_per_tile,), jnp.int32),
            pltpu.VMEM((rows_per_tile, _NUM_UNITS), jnp.float32),
            pltpu.VMEM_SHARED((e_rows, _NUM_UNITS), jnp.float32),
            pltpu.SemaphoreType.DMA,
        ],
        compiler_params=pltpu.CompilerParams(use_tc_tiling_on_sc=False),
    )
    def k(table_hbm, rv_hbm, out_hbm, rv_v, idx_v, rows_v, e_sh, sem):
        c = lax.axis_index("c")
        s = lax.axis_index("s")

        pltpu.sync_copy(rv_hbm, rv_v)
        rv_vec = rv_v[...]
        row0 = s * rows_per_tile
        for ch in range(rows_per_tile // 16):
            kk = lax.iota(jnp.int32, 16) + (row0 + ch * 16)
            idx = jnp.clip(kk - (_L - 1), -_MAX_REL, _MAX_REL) + _MAX_REL + rv_vec
            idx = jnp.clip(idx, 0, rows - 1)
            idx_v[pl.ds(ch * 16, 16)] = idx
        for g in range(rows_per_tile // 128):
            pltpu.async_copy(
                table_hbm.at[idx_v.at[pl.ds(g * 128, 128)]],
                rows_v.at[pl.ds(g * 128, 128)],
                sem,
            ).wait()
        pltpu.sync_copy(rows_v, e_sh.at[pl.ds(row0, rows_per_tile)])
        plsc.subcore_barrier()

        base = (c * ns + s) * out_rows_per_worker

        def body(t, carry):
            i = base + t
            pltpu.sync_copy(e_sh.at[pl.ds((_L - 1) - i, _L)], out_hbm.at[i])
            return carry

        lax.fori_loop(0, out_rows_per_worker, body, 0)

    return k(table, rv_arr)


def kernel(embeddings_table, length_q, length_k, relative_v):
    return _rel_pos_sc(embeddings_table, relative_v)
